# Initial kernel scaffold; baseline (speedup 1.0000x reference)
import jax
import jax.numpy as jnp
from jax.experimental import pallas as pl


def kernel(x_n0, x_n1, params, edge_index_n0_n1, edge_index_n1_n0, batch_n0, batch_n1):
    raise NotImplementedError("write your pallas kernel here")


# speedup vs baseline: 7.1243x; 7.1243x over previous
import functools

import jax
import jax.numpy as jnp
from jax import lax
from jax.experimental import pallas as pl
from jax.experimental.pallas import tpu as pltpu
from jax.experimental.pallas import tpu_sc as plsc

f32 = jnp.float32
i32 = jnp.int32

H = 128
N = 25000
NP = 25600
G = 64
E = 400000
EP = 409600
L = 3

NTILES = 16
NCORES = 2
HALF = NP // 2
JUNK = 64
ACC_ROWS = HALF + JUNK
ZROWS = ACC_ROWS // NTILES
WROWS = HALF // NTILES
SHARE = EP // NTILES
CSTEPS = SHARE // 16
CHUNK = 64
CAP = 26112
NCHUNK = CAP // CHUNK
FSTEPS = CAP // 16
IB = 24
NBLK = NCHUNK // IB
BC = 1600


@functools.cache
def _sc_kernels():
    mesh = plsc.VectorSubcoreMesh(
        core_axis_name="c", subcore_axis_name="s",
        num_cores=NCORES, num_subcores=NTILES)

    @functools.partial(
        pl.kernel,
        out_type=(jax.ShapeDtypeStruct((NCORES * NTILES * CAP,), i32),
                  jax.ShapeDtypeStruct((NCORES * NTILES * CAP,), i32),
                  jax.ShapeDtypeStruct((NCORES * NTILES * 16,), i32)),
        mesh=mesh,
        compiler_params=pltpu.CompilerParams(needs_layout_passes=False),
        scratch_types=[
            pltpu.VMEM((SHARE,), i32),
            pltpu.VMEM((SHARE,), i32),
            pltpu.VMEM((CAP,), i32),
            pltpu.VMEM((CAP,), i32),
            pltpu.VMEM((16,), i32),
        ],
    )
    def _partition(src_hbm, dst_hbm, srcp_hbm, dstp_hbm, cnt_hbm,
                   src_in, dst_in, src_out, dst_out, cnt_v):
        c = lax.axis_index("c")
        s = lax.axis_index("s")
        tid = c * NTILES + s
        pltpu.sync_copy(src_hbm.at[pl.ds(s * SHARE, SHARE)], src_in)
        pltpu.sync_copy(dst_hbm.at[pl.ds(s * SHARE, SHARE)], dst_in)
        lo = c * HALF

        def prefill(i, carry):
            lanes = lax.iota(i32, 16) + i * 16
            src_out[pl.ds(i * 16, 16)] = lanes & 16383
            dst_out[pl.ds(i * 16, 16)] = HALF + (lanes & (JUNK - 1))
            return carry

        lax.fori_loop(0, FSTEPS, prefill, 0)

        def compact(i, off):
            d = dst_in[pl.ds(i * 16, 16)]
            sv = src_in[pl.ds(i * 16, 16)]
            dl = d - lo
            m = (dl >= 0) & (dl < HALF)
            plsc.store_compressed(src_out.at[pl.ds(off, 16)], sv, mask=m)
            plsc.store_compressed(dst_out.at[pl.ds(off, 16)], dl, mask=m)
            return off + jnp.sum(m.astype(i32))

        off = lax.fori_loop(0, CSTEPS, compact, jnp.zeros((), i32))
        nchunks = 3 * ((off + 3 * CHUNK - 1) // (3 * CHUNK))
        cnt_v[...] = jnp.broadcast_to(nchunks, (16,)).astype(i32)
        pltpu.sync_copy(src_out, srcp_hbm.at[pl.ds(tid * CAP, CAP)])
        pltpu.sync_copy(dst_out, dstp_hbm.at[pl.ds(tid * CAP, CAP)])
        pltpu.sync_copy(cnt_v, cnt_hbm.at[pl.ds(tid * 16, 16)])

    @functools.partial(
        pl.kernel,
        out_type=jax.ShapeDtypeStruct((NP, H), f32),
        mesh=mesh,
        compiler_params=pltpu.CompilerParams(needs_layout_passes=False),
        scratch_types=[
            pltpu.VMEM((IB * CHUNK,), i32),
            pltpu.VMEM((IB * CHUNK,), i32),
            pltpu.VMEM((CHUNK, H), f32),
            pltpu.VMEM((CHUNK, H), f32),
            pltpu.VMEM((CHUNK, H), f32),
            pltpu.VMEM((16,), i32),
            pltpu.VMEM_SHARED((ACC_ROWS, H), f32),
            pltpu.SemaphoreType.DMA,
            pltpu.SemaphoreType.DMA,
            pltpu.SemaphoreType.DMA,
            pltpu.SemaphoreType.DMA,
            pltpu.SemaphoreType.DMA,
            pltpu.SemaphoreType.DMA,
        ],
    )
    def _segsum(x_hbm, srcp_hbm, dstp_hbm, cnt_hbm, zeros_hbm, out_hbm,
                src_v, dst_v, rows_a, rows_b, rows_c, cnt_v, acc,
                gsa, gsb, gsc, ssa, ssb, ssc):
        c = lax.axis_index("c")
        s = lax.axis_index("s")
        tid = c * NTILES + s
        pltpu.sync_copy(zeros_hbm, acc.at[pl.ds(s * ZROWS, ZROWS)])
        pltpu.sync_copy(cnt_hbm.at[pl.ds(tid * 16, 16)], cnt_v)
        plsc.subcore_barrier()
        n = cnt_v[...][0]

        def _dst(j):
            return dst_v.at[pl.ds(j * CHUNK, CHUNK)]

        def _gather(j, rows, sem):
            return pltpu.async_copy(
                x_hbm.at[src_v.at[pl.ds(j * CHUNK, CHUNK)]], rows, sem)

        def _scatter(j, rows, sem):
            return pltpu.async_copy(rows, acc.at[_dst(j)], sem, add=True)

        def _gwait(rows, sem):
            pltpu.make_async_copy(x_hbm.at[_dst(0)], rows, sem).wait()

        def _swait(rows, sem):
            pltpu.make_async_copy(rows, acc.at[_dst(0)], sem).wait()

        def blk_body(b, carry):
            base = tid * CAP + b * IB * CHUNK
            pltpu.sync_copy(srcp_hbm.at[pl.ds(base, IB * CHUNK)], src_v)
            pltpu.sync_copy(dstp_hbm.at[pl.ds(base, IB * CHUNK)], dst_v)
            nt = jnp.minimum(n - b * IB, IB) // 3

            _gather(0, rows_a, gsa)
            _gather(1, rows_b, gsb)

            def triple(t, inner):
                _gwait(rows_a, gsa)
                _scatter(3 * t, rows_a, ssa)

                @pl.when(t > 0)
                def _():
                    _swait(rows_c, ssc)
                _gather(3 * t + 2, rows_c, gsc)
                _gwait(rows_b, gsb)
                _scatter(3 * t + 1, rows_b, ssb)
                _swait(rows_a, ssa)

                @pl.when(t < nt - 1)
                def _():
                    _gather(3 * t + 3, rows_a, gsa)
                _gwait(rows_c, gsc)
                _scatter(3 * t + 2, rows_c, ssc)
                _swait(rows_b, ssb)

                @pl.when(t < nt - 1)
                def _():
                    _gather(3 * t + 4, rows_b, gsb)
                return inner

            lax.fori_loop(0, nt, triple, 0)
            _swait(rows_c, ssc)
            return carry

        lax.fori_loop(0, (n + IB - 1) // IB, blk_body, 0)
        plsc.subcore_barrier()
        pltpu.sync_copy(acc.at[pl.ds(s * WROWS, WROWS)],
                        out_hbm.at[pl.ds(c * HALF + s * WROWS, WROWS)])

    return _partition, _segsum


def _partition_sc(src, dst):
    return _sc_kernels()[0](src, dst)


def _segsum_sc(x, srcp, dstp, cnt, zeros):
    return _sc_kernels()[1](x, srcp, dstp, cnt, zeros)


def _conv_body(relu, agg_ref, h_ref, wrel_ref, wroot_ref, b_ref, out_ref):
    res = (jnp.dot(agg_ref[...], wrel_ref[...], preferred_element_type=f32)
           + jnp.dot(h_ref[...], wroot_ref[...], preferred_element_type=f32)
           + b_ref[...])
    if relu:
        res = jnp.maximum(res, 0.0)
    out_ref[...] = res


def _conv(agg, h, wrel, wroot, b, relu):
    return pl.pallas_call(
        functools.partial(_conv_body, relu),
        grid=(NP // BC,),
        in_specs=[
            pl.BlockSpec((BC, H), lambda i: (i, 0)),
            pl.BlockSpec((BC, H), lambda i: (i, 0)),
            pl.BlockSpec((H, H), lambda i: (0, 0)),
            pl.BlockSpec((H, H), lambda i: (0, 0)),
            pl.BlockSpec((1, H), lambda i: (0, 0)),
        ],
        out_specs=pl.BlockSpec((BC, H), lambda i: (i, 0)),
        out_shape=jax.ShapeDtypeStruct((NP, H), f32),
    )(agg, h, wrel, wroot, b)


def _pool_body(h0_ref, h1_ref, b0_ref, b1_ref, lo0, hi0, lo1, hi1,
               w1a, b1a, w2a, b2a, w1b, b1b, w2b, b2b, w0v, w1v, bov,
               out_ref, acc0, acc1):
    i = pl.program_id(0)
    nsteps = pl.num_programs(0)

    @pl.when(i == 0)
    def _():
        acc0[...] = jnp.full((G, H), -jnp.inf, f32)
        acc1[...] = jnp.full((G, H), -jnp.inf, f32)

    def accumulate(h_ref, b_ref, acc, lo, hi):
        val = h_ref[...]
        b = b_ref[...]

        def gbody(g, carry):
            v = jnp.max(jnp.where(b == g, val, -jnp.inf), axis=0,
                        keepdims=True)
            acc[pl.ds(g, 1), :] = jnp.maximum(acc[pl.ds(g, 1), :], v)
            return carry

        lax.fori_loop(lo, hi + 1, gbody, 0)

    accumulate(h0_ref, b0_ref, acc0, lo0[i], hi0[i])
    accumulate(h1_ref, b1_ref, acc1, lo1[i], hi1[i])

    @pl.when(i == nsteps - 1)
    def _():
        t0 = jnp.maximum(
            jnp.dot(acc0[...], w1a[...], preferred_element_type=f32) + b1a[...],
            0.0)
        m0 = jnp.dot(t0, w2a[...], preferred_element_type=f32) + b2a[...]
        t1 = jnp.maximum(
            jnp.dot(acc1[...], w1b[...], preferred_element_type=f32) + b1b[...],
            0.0)
        m1 = jnp.dot(t1, w2b[...], preferred_element_type=f32) + b2b[...]
        out_ref[...] = m0 * w0v[...] + m1 * w1v[...] + bov[...]


def _pool_mlp(h0, h1, b0c, b1c, lo0, hi0, lo1, hi1, mlp0, mlp1, wouts):
    full = pl.BlockSpec((H, H), lambda i: (0, 0))
    row = pl.BlockSpec((1, H), lambda i: (0, 0))
    smem = pl.BlockSpec(memory_space=pltpu.SMEM)
    out = pl.pallas_call(
        _pool_body,
        grid=(NP // BC,),
        in_specs=[
            pl.BlockSpec((BC, H), lambda i: (i, 0)),
            pl.BlockSpec((BC, H), lambda i: (i, 0)),
            pl.BlockSpec((BC, 1), lambda i: (i, 0)),
            pl.BlockSpec((BC, 1), lambda i: (i, 0)),
            smem, smem, smem, smem,
            full, row, full, row,
            full, row, full, row,
            row, row, row,
        ],
        out_specs=pl.BlockSpec((G, H), lambda i: (0, 0)),
        out_shape=jax.ShapeDtypeStruct((G, H), f32),
        scratch_shapes=[pltpu.VMEM((G, H), f32), pltpu.VMEM((G, H), f32)],
    )(h0, h1, b0c, b1c, lo0, hi0, lo1, hi1, *mlp0, *mlp1, *wouts)
    return lax.slice(out, (0, 0), (G, 1))


def _prep_edges(ei):
    pad = EP - E
    pi = jnp.arange(pad, dtype=i32)
    src = jnp.concatenate([ei[0], pi % N])
    dst = jnp.concatenate([ei[1], N + pi % (NP - N)])
    return src, dst


def _prep_batch(batch):
    bp = jnp.concatenate([batch, jnp.full((NP - N,), i32(G + 1), i32)])
    blk = bp.reshape(NP // BC, BC)
    lo = blk[:, 0]
    hi = jnp.minimum(blk[:, -1], G - 1)
    return bp.reshape(NP, 1), lo, hi


def _pad_mlp(p):
    w1 = jnp.zeros((H, H), f32).at[:, :5].set(p['W1'])
    b1 = jnp.zeros((1, H), f32).at[0, :5].set(p['b1'])
    w2 = jnp.zeros((H, H), f32).at[:5, 0].set(p['W2'][:, 0])
    b2 = jnp.broadcast_to(p['b2'].reshape(1, 1), (1, H))
    return w1, b1, w2, b2


def kernel(x_n0, x_n1, params, edge_index_n0_n1, edge_index_n1_n0,
           batch_n0, batch_n1):
    zeros_tile = jnp.zeros((ZROWS, H), f32)
    src01, dst01 = _prep_edges(edge_index_n0_n1)
    src10, dst10 = _prep_edges(edge_index_n1_n0)
    srcp01, dstp01, cnt01 = _partition_sc(src01, dst01)
    srcp10, dstp10, cnt10 = _partition_sc(src10, dst10)

    h0 = jnp.pad(x_n0, ((0, NP - N), (0, 0)))
    h1 = jnp.pad(x_n1, ((0, NP - N), (0, 0)))

    for l in range(L):
        p01 = params['conv01_' + str(l)]
        p10 = params['conv10_' + str(l)]
        agg1 = _segsum_sc(h0, srcp01, dstp01, cnt01, zeros_tile)
        agg0 = _segsum_sc(h1, srcp10, dstp10, cnt10, zeros_tile)
        relu = l < L - 1
        h1n = _conv(agg1, h1, p01['W_rel'], p01['W_root'],
                    p01['b_rel'].reshape(1, H), relu)
        h0n = _conv(agg0, h0, p10['W_rel'], p10['W_root'],
                    p10['b_rel'].reshape(1, H), relu)
        h0, h1 = h0n, h1n

    b0c, lo0, hi0 = _prep_batch(batch_n0)
    b1c, lo1, hi1 = _prep_batch(batch_n1)
    mlp0 = _pad_mlp(params['mlp_n0'])
    mlp1 = _pad_mlp(params['mlp_n1'])
    wout = params['Wout']
    wouts = (jnp.broadcast_to(wout[0].reshape(1, 1), (1, H)),
             jnp.broadcast_to(wout[1].reshape(1, 1), (1, H)),
             jnp.broadcast_to(params['bout'].reshape(1, 1), (1, H)))
    return _pool_mlp(h0, h1, b0c, b1c, lo0, hi0, lo1, hi1, mlp0, mlp1, wouts)


# speedup vs baseline: 7.8714x; 1.1049x over previous
import functools

import jax
import jax.numpy as jnp
from jax import lax
from jax.experimental import pallas as pl
from jax.experimental.pallas import tpu as pltpu
from jax.experimental.pallas import tpu_sc as plsc

f32 = jnp.float32
i32 = jnp.int32

H = 128
N = 25000
NP = 25600
G = 64
E = 400000
EP = 400128
L = 3

NTILES = 16
NCORES = 2
HALF = NP // 2
JUNK = 64
ACC_ROWS = HALF + JUNK
ZROWS = ACC_ROWS // NTILES
WROWS = HALF // NTILES
SHARE = EP // NTILES
CSTEPS = SHARE // 16
CHUNK = 64
CAP = 26112
NCHUNK = CAP // CHUNK
FSTEPS = CAP // 16
IB = 24
NBLK = NCHUNK // IB
BC = 1600


@functools.cache
def _sc_kernels():
    mesh = plsc.VectorSubcoreMesh(
        core_axis_name="c", subcore_axis_name="s",
        num_cores=NCORES, num_subcores=NTILES)

    @functools.partial(
        pl.kernel,
        out_type=(jax.ShapeDtypeStruct((NCORES * NTILES * CAP,), i32),
                  jax.ShapeDtypeStruct((NCORES * NTILES * CAP,), i32),
                  jax.ShapeDtypeStruct((NCORES * NTILES * CAP,), i32),
                  jax.ShapeDtypeStruct((NCORES * NTILES * CAP,), i32),
                  jax.ShapeDtypeStruct((2 * NCORES * NTILES * 16,), i32)),
        mesh=mesh,
        compiler_params=pltpu.CompilerParams(needs_layout_passes=False),
        scratch_types=[
            pltpu.VMEM((SHARE,), i32),
            pltpu.VMEM((SHARE,), i32),
            pltpu.VMEM((CAP,), i32),
            pltpu.VMEM((CAP,), i32),
            pltpu.VMEM((16,), i32),
        ],
    )
    def _partition(srcA_hbm, dstA_hbm, srcB_hbm, dstB_hbm,
                   srcpA_hbm, dstpA_hbm, srcpB_hbm, dstpB_hbm, cnt_hbm,
                   src_in, dst_in, src_out, dst_out, cnt_v):
        c = lax.axis_index("c")
        s = lax.axis_index("s")
        tid = c * NTILES + s
        lo = c * HALF

        def prefill(i, carry):
            lanes = lax.iota(i32, 16) + i * 16
            src_out[pl.ds(i * 16, 16)] = lanes & 16383
            dst_out[pl.ds(i * 16, 16)] = HALF + (lanes & (JUNK - 1))
            return carry

        def compact(i, off):
            d = dst_in[pl.ds(i * 16, 16)]
            sv = src_in[pl.ds(i * 16, 16)]
            dl = d - lo
            m = (dl >= 0) & (dl < HALF)
            plsc.store_compressed(src_out.at[pl.ds(off, 16)], sv, mask=m)
            plsc.store_compressed(dst_out.at[pl.ds(off, 16)], dl, mask=m)
            return off + jnp.sum(m.astype(i32))

        def run_dir(src_hbm, dst_hbm, srcp_hbm, dstp_hbm, slot):
            pltpu.sync_copy(src_hbm.at[pl.ds(s * SHARE, SHARE)], src_in)
            pltpu.sync_copy(dst_hbm.at[pl.ds(s * SHARE, SHARE)], dst_in)
            lax.fori_loop(0, FSTEPS, prefill, 0)
            off = lax.fori_loop(0, CSTEPS, compact, jnp.zeros((), i32))
            nchunks = 3 * ((off + 3 * CHUNK - 1) // (3 * CHUNK))
            cnt_v[...] = jnp.broadcast_to(nchunks, (16,)).astype(i32)
            pltpu.sync_copy(src_out, srcp_hbm.at[pl.ds(tid * CAP, CAP)])
            pltpu.sync_copy(dst_out, dstp_hbm.at[pl.ds(tid * CAP, CAP)])
            pltpu.sync_copy(cnt_v,
                            cnt_hbm.at[pl.ds((slot * 32 + tid) * 16, 16)])

        run_dir(srcA_hbm, dstA_hbm, srcpA_hbm, dstpA_hbm, 0)
        run_dir(srcB_hbm, dstB_hbm, srcpB_hbm, dstpB_hbm, 1)

    @functools.partial(
        pl.kernel,
        out_type=(jax.ShapeDtypeStruct((NP, H), f32),
                  jax.ShapeDtypeStruct((NP, H), f32)),
        mesh=mesh,
        compiler_params=pltpu.CompilerParams(needs_layout_passes=False),
        scratch_types=[
            pltpu.VMEM((IB * CHUNK,), i32),
            pltpu.VMEM((IB * CHUNK,), i32),
            pltpu.VMEM((CHUNK, H), f32),
            pltpu.VMEM((CHUNK, H), f32),
            pltpu.VMEM((CHUNK, H), f32),
            pltpu.VMEM((16,), i32),
            pltpu.VMEM_SHARED((ACC_ROWS, H), f32),
            pltpu.SemaphoreType.DMA,
            pltpu.SemaphoreType.DMA,
            pltpu.SemaphoreType.DMA,
            pltpu.SemaphoreType.DMA,
            pltpu.SemaphoreType.DMA,
            pltpu.SemaphoreType.DMA,
        ],
    )
    def _segsum(xA_hbm, srcpA_hbm, dstpA_hbm, xB_hbm, srcpB_hbm, dstpB_hbm,
                cnt_hbm, zeros_hbm, outA_hbm, outB_hbm,
                src_v, dst_v, rows_a, rows_b, rows_c, cnt_v, acc,
                gsa, gsb, gsc, ssa, ssb, ssc):
        c = lax.axis_index("c")
        s = lax.axis_index("s")
        tid = c * NTILES + s

        def _dst(j):
            return dst_v.at[pl.ds(j * CHUNK, CHUNK)]

        def run_dir(x_hbm, srcp_hbm, dstp_hbm, out_hbm, slot):
            pltpu.sync_copy(zeros_hbm,
                            acc.at[pl.ds(s * ZROWS, ZROWS)])
            pltpu.sync_copy(cnt_hbm.at[pl.ds((slot * 32 + tid) * 16, 16)],
                            cnt_v)
            plsc.subcore_barrier()
            n = cnt_v[...][0]

            def _gather(j, rows, sem):
                return pltpu.async_copy(
                    x_hbm.at[src_v.at[pl.ds(j * CHUNK, CHUNK)]], rows, sem)

            def _scatter(j, rows, sem):
                return pltpu.async_copy(rows, acc.at[_dst(j)], sem, add=True)

            def _gwait(rows, sem):
                pltpu.make_async_copy(x_hbm.at[_dst(0)], rows, sem).wait()

            def _swait(rows, sem):
                pltpu.make_async_copy(rows, acc.at[_dst(0)], sem).wait()

            def blk_body(b, carry):
                base = tid * CAP + b * IB * CHUNK
                pltpu.sync_copy(srcp_hbm.at[pl.ds(base, IB * CHUNK)], src_v)
                pltpu.sync_copy(dstp_hbm.at[pl.ds(base, IB * CHUNK)], dst_v)
                nt = jnp.minimum(n - b * IB, IB) // 3

                _gather(0, rows_a, gsa)
                _gather(1, rows_b, gsb)

                def triple(t, inner):
                    _gwait(rows_a, gsa)
                    _scatter(3 * t, rows_a, ssa)

                    @pl.when(t > 0)
                    def _():
                        _swait(rows_c, ssc)
                    _gather(3 * t + 2, rows_c, gsc)
                    _gwait(rows_b, gsb)
                    _scatter(3 * t + 1, rows_b, ssb)
                    _swait(rows_a, ssa)

                    @pl.when(t < nt - 1)
                    def _():
                        _gather(3 * t + 3, rows_a, gsa)
                    _gwait(rows_c, gsc)
                    _scatter(3 * t + 2, rows_c, ssc)
                    _swait(rows_b, ssb)

                    @pl.when(t < nt - 1)
                    def _():
                        _gather(3 * t + 4, rows_b, gsb)
                    return inner

                lax.fori_loop(0, nt, triple, 0)
                _swait(rows_c, ssc)
                return carry

            lax.fori_loop(0, (n + IB - 1) // IB, blk_body, 0)
            plsc.subcore_barrier()
            pltpu.sync_copy(acc.at[pl.ds(s * WROWS, WROWS)],
                            out_hbm.at[pl.ds(c * HALF + s * WROWS, WROWS)])
            plsc.subcore_barrier()

        run_dir(xA_hbm, srcpA_hbm, dstpA_hbm, outA_hbm, 0)
        run_dir(xB_hbm, srcpB_hbm, dstpB_hbm, outB_hbm, 1)

    return _partition, _segsum


def _partition_sc(srcA, dstA, srcB, dstB):
    return _sc_kernels()[0](srcA, dstA, srcB, dstB)


def _segsum_sc(xA, srcpA, dstpA, xB, srcpB, dstpB, cnt, zeros):
    return _sc_kernels()[1](xA, srcpA, dstpA, xB, srcpB, dstpB, cnt, zeros)


def _conv_body(relu, agg_ref, h_ref, wrel_ref, wroot_ref, b_ref, out_ref):
    res = (jnp.dot(agg_ref[...], wrel_ref[...], preferred_element_type=f32)
           + jnp.dot(h_ref[...], wroot_ref[...], preferred_element_type=f32)
           + b_ref[...])
    if relu:
        res = jnp.maximum(res, 0.0)
    out_ref[...] = res


def _conv(agg, h, wrel, wroot, b, relu):
    return pl.pallas_call(
        functools.partial(_conv_body, relu),
        grid=(NP // BC,),
        in_specs=[
            pl.BlockSpec((BC, H), lambda i: (i, 0)),
            pl.BlockSpec((BC, H), lambda i: (i, 0)),
            pl.BlockSpec((H, H), lambda i: (0, 0)),
            pl.BlockSpec((H, H), lambda i: (0, 0)),
            pl.BlockSpec((1, H), lambda i: (0, 0)),
        ],
        out_specs=pl.BlockSpec((BC, H), lambda i: (i, 0)),
        out_shape=jax.ShapeDtypeStruct((NP, H), f32),
    )(agg, h, wrel, wroot, b)


def _pool_body(h0_ref, h1_ref, b0_ref, b1_ref, lo0, hi0, lo1, hi1,
               w1a, b1a, w2a, b2a, w1b, b1b, w2b, b2b, w0v, w1v, bov,
               out_ref, acc0, acc1):
    i = pl.program_id(0)
    nsteps = pl.num_programs(0)

    @pl.when(i == 0)
    def _():
        acc0[...] = jnp.full((G, H), -jnp.inf, f32)
        acc1[...] = jnp.full((G, H), -jnp.inf, f32)

    def accumulate(h_ref, b_ref, acc, lo, hi):
        val = h_ref[...]
        b = b_ref[...]

        def gbody(g, carry):
            v = jnp.max(jnp.where(b == g, val, -jnp.inf), axis=0,
                        keepdims=True)
            acc[pl.ds(g, 1), :] = jnp.maximum(acc[pl.ds(g, 1), :], v)
            return carry

        lax.fori_loop(lo, hi + 1, gbody, 0)

    accumulate(h0_ref, b0_ref, acc0, lo0[i], hi0[i])
    accumulate(h1_ref, b1_ref, acc1, lo1[i], hi1[i])

    @pl.when(i == nsteps - 1)
    def _():
        t0 = jnp.maximum(
            jnp.dot(acc0[...], w1a[...], preferred_element_type=f32) + b1a[...],
            0.0)
        m0 = jnp.dot(t0, w2a[...], preferred_element_type=f32) + b2a[...]
        t1 = jnp.maximum(
            jnp.dot(acc1[...], w1b[...], preferred_element_type=f32) + b1b[...],
            0.0)
        m1 = jnp.dot(t1, w2b[...], preferred_element_type=f32) + b2b[...]
        out_ref[...] = m0 * w0v[...] + m1 * w1v[...] + bov[...]


def _pool_mlp(h0, h1, b0c, b1c, lo0, hi0, lo1, hi1, mlp0, mlp1, wouts):
    full = pl.BlockSpec((H, H), lambda i: (0, 0))
    row = pl.BlockSpec((1, H), lambda i: (0, 0))
    smem = pl.BlockSpec(memory_space=pltpu.SMEM)
    out = pl.pallas_call(
        _pool_body,
        grid=(NP // BC,),
        in_specs=[
            pl.BlockSpec((BC, H), lambda i: (i, 0)),
            pl.BlockSpec((BC, H), lambda i: (i, 0)),
            pl.BlockSpec((BC, 1), lambda i: (i, 0)),
            pl.BlockSpec((BC, 1), lambda i: (i, 0)),
            smem, smem, smem, smem,
            full, row, full, row,
            full, row, full, row,
            row, row, row,
        ],
        out_specs=pl.BlockSpec((G, H), lambda i: (0, 0)),
        out_shape=jax.ShapeDtypeStruct((G, H), f32),
        scratch_shapes=[pltpu.VMEM((G, H), f32), pltpu.VMEM((G, H), f32)],
    )(h0, h1, b0c, b1c, lo0, hi0, lo1, hi1, *mlp0, *mlp1, *wouts)
    return lax.slice(out, (0, 0), (G, 1))


def _prep_edges(ei):
    pad = EP - E
    pi = jnp.arange(pad, dtype=i32)
    src = jnp.concatenate([ei[0], pi % N])
    dst = jnp.concatenate([ei[1], N + pi % (NP - N)])
    return src, dst


def _prep_batch(batch):
    bp = jnp.concatenate([batch, jnp.full((NP - N,), i32(G + 1), i32)])
    blk = bp.reshape(NP // BC, BC)
    lo = blk[:, 0]
    hi = jnp.minimum(blk[:, -1], G - 1)
    return bp.reshape(NP, 1), lo, hi


def _pad_mlp(p):
    w1 = jnp.zeros((H, H), f32).at[:, :5].set(p['W1'])
    b1 = jnp.zeros((1, H), f32).at[0, :5].set(p['b1'])
    w2 = jnp.zeros((H, H), f32).at[:5, 0].set(p['W2'][:, 0])
    b2 = jnp.broadcast_to(p['b2'].reshape(1, 1), (1, H))
    return w1, b1, w2, b2


def kernel(x_n0, x_n1, params, edge_index_n0_n1, edge_index_n1_n0,
           batch_n0, batch_n1):
    zeros_tile = jnp.zeros((ZROWS, H), f32)
    src01, dst01 = _prep_edges(edge_index_n0_n1)
    src10, dst10 = _prep_edges(edge_index_n1_n0)
    srcp01, dstp01, srcp10, dstp10, cnt = _partition_sc(
        src01, dst01, src10, dst10)

    h0 = jnp.pad(x_n0, ((0, NP - N), (0, 0)))
    h1 = jnp.pad(x_n1, ((0, NP - N), (0, 0)))

    for l in range(L):
        p01 = params['conv01_' + str(l)]
        p10 = params['conv10_' + str(l)]
        agg1, agg0 = _segsum_sc(h0, srcp01, dstp01, h1, srcp10, dstp10,
                                cnt, zeros_tile)
        relu = l < L - 1
        h1n = _conv(agg1, h1, p01['W_rel'], p01['W_root'],
                    p01['b_rel'].reshape(1, H), relu)
        h0n = _conv(agg0, h0, p10['W_rel'], p10['W_root'],
                    p10['b_rel'].reshape(1, H), relu)
        h0, h1 = h0n, h1n

    b0c, lo0, hi0 = _prep_batch(batch_n0)
    b1c, lo1, hi1 = _prep_batch(batch_n1)
    mlp0 = _pad_mlp(params['mlp_n0'])
    mlp1 = _pad_mlp(params['mlp_n1'])
    wout = params['Wout']
    wouts = (jnp.broadcast_to(wout[0].reshape(1, 1), (1, H)),
             jnp.broadcast_to(wout[1].reshape(1, 1), (1, H)),
             jnp.broadcast_to(params['bout'].reshape(1, 1), (1, H)))
    return _pool_mlp(h0, h1, b0c, b1c, lo0, hi0, lo1, hi1, mlp0, mlp1, wouts)
